# baseline (speedup 1.0000x reference)
import functools

import jax
import jax.numpy as jnp
from jax import lax
from jax.experimental import pallas as pl
from jax.experimental.pallas import tpu as pltpu
from jax.experimental.pallas import tpu_sc as plsc

_LEVELS = 256
_D = 8192
_B = 4096
_NW = 32
_ROWS_PER_W = _B // _NW
_QD = _D // 4
_G = 2


def _prep(x, w):
    def body(x_ref, w_ref, t_ref, idx_ref):
        wv = w_ref[...]
        t_ref[pl.ds(0, _LEVELS), :] = jnp.concatenate(
            [wv[:, -2:], wv[:, :-2]], axis=1)
        t_ref[pl.ds(_LEVELS, _LEVELS), :] = jnp.concatenate(
            [wv[:, -1:], wv[:, :-1]], axis=1)
        t_ref[pl.ds(2 * _LEVELS, _LEVELS), :] = wv

        xv = x_ref[...]
        q = jnp.clip(jnp.round(xv * (_LEVELS - 1)).astype(jnp.int32),
                     0, _LEVELS - 1)
        jc = lax.broadcasted_iota(jnp.int32, (_B, 12), 1)
        ch = jc // 4
        qt = jc % 4
        sel = jnp.where(ch == 0, q[:, 0:1],
                        jnp.where(ch == 1, q[:, 1:2], q[:, 2:3]))
        idx_ref[...] = ch * (4 * _LEVELS) + sel * 4 + qt

    t, idx = pl.pallas_call(
        body,
        out_shape=[
            jax.ShapeDtypeStruct((3 * _LEVELS, _D), jnp.float32),
            jax.ShapeDtypeStruct((_B, 12), jnp.int32),
        ],
    )(x, w)
    return t.reshape(3 * _LEVELS * 4, _QD), idx.reshape(-1)


def _sc_encode(t4, idx_flat):
    mesh = plsc.VectorSubcoreMesh(
        core_axis_name="c", subcore_axis_name="s", num_cores=2, num_subcores=16)

    @functools.partial(
        pl.kernel,
        mesh=mesh,
        out_type=jax.ShapeDtypeStruct((_B, _D), jnp.float32),
        scratch_types=[
            pltpu.VMEM((_ROWS_PER_W * 12,), jnp.int32),
            pltpu.VMEM((12 * _G, _QD), jnp.float32),
            pltpu.VMEM((_G, _D), jnp.float32),
            pltpu.SemaphoreType.DMA,
        ],
    )
    def k(t4_hbm, idx_hbm, out_hbm, idx_v, rows_v, out_v, sem):
        wid = lax.axis_index("s") * 2 + lax.axis_index("c")
        base = wid * _ROWS_PER_W
        pltpu.sync_copy(idx_hbm.at[pl.ds(base * 12, _ROWS_PER_W * 12)], idx_v)

        def group(g, carry):
            pltpu.async_copy(
                t4_hbm.at[idx_v.at[pl.ds(g * (12 * _G), 12 * _G)]],
                rows_v, sem).wait()
            for p in range(_G):
                for q in range(4):
                    r0 = 12 * p + q

                    def feat(v, c, r0=r0, p=p, q=q):
                        a = rows_v[r0, pl.ds(v * 16, 16)]
                        b = rows_v[r0 + 4, pl.ds(v * 16, 16)]
                        d = rows_v[r0 + 8, pl.ds(v * 16, 16)]
                        prod = a * b * d
                        out_v[p, pl.ds(q * _QD + v * 16, 16)] = jnp.where(
                            prod > 0, 1.0, -1.0)
                        return c

                    lax.fori_loop(0, _QD // 16, feat, 0)
            pltpu.sync_copy(out_v, out_hbm.at[pl.ds(base + g * _G, _G)])
            return carry

        lax.fori_loop(0, _ROWS_PER_W // _G, group, 0)

    return k(t4, idx_flat)


def kernel(x, level_weight):
    t4, idx_flat = _prep(x, level_weight)
    return _sc_encode(t4, idx_flat)


# speedup vs baseline: 8.1202x; 8.1202x over previous
import functools

import numpy as np
import jax
import jax.numpy as jnp
from jax import lax
from jax.experimental import pallas as pl
from jax.experimental.pallas import tpu as pltpu
from jax.experimental.pallas import tpu_sc as plsc

_LEVELS = 256
_D = 8192
_B = 4096
_NW = 32
_ROWS_PER_W = _B // _NW
_WPR = _D // 32
_G = 8
_NGROUPS = _ROWS_PER_W // _G
_HALF = _G // 2


def _pack_matrix():
    f = np.arange(_D)
    block = f // 512
    k = (f % 512) // 16
    lane = f % 16
    col = np.where(k < 16, block * 16 + lane, _WPR + block * 16 + lane)
    m = np.zeros((_D, 2 * _WPR), np.float32)
    m[f, col] = 2.0 ** (k % 16)
    return m


_M_NP = _pack_matrix()


def _prep(x, w):
    def body(x_ref, w_ref, m_ref, pk_ref, idx_ref):
        b0 = (0.5 - 0.5 * w_ref[...]).astype(jnp.bfloat16)
        m = m_ref[...]
        for c, sh in enumerate((2, 1, 0)):
            bc = b0 if sh == 0 else jnp.concatenate(
                [b0[:, -sh:], b0[:, :-sh]], axis=1)
            h = jnp.dot(bc, m, preferred_element_type=jnp.float32)
            lo = h[:, :_WPR].astype(jnp.int32)
            hi = h[:, _WPR:].astype(jnp.int32)
            pk_ref[pl.ds(c * _LEVELS, _LEVELS), :] = lo | (hi << 16)

        xv = x_ref[...]
        q = jnp.clip(jnp.round(xv * (_LEVELS - 1)).astype(jnp.int32),
                     0, _LEVELS - 1)
        ch = lax.broadcasted_iota(jnp.int32, (_B, 3), 1)
        idx_ref[...] = q + ch * _LEVELS

    pk, idx = pl.pallas_call(
        body,
        out_shape=[
            jax.ShapeDtypeStruct((3 * _LEVELS, _WPR), jnp.int32),
            jax.ShapeDtypeStruct((_B, 3), jnp.int32),
        ],
    )(x, w, jnp.asarray(_M_NP, dtype=jnp.bfloat16))
    return pk, idx.reshape(-1)


def _sc_encode(pk_tab, idx_flat):
    mesh = plsc.VectorSubcoreMesh(
        core_axis_name="c", subcore_axis_name="s", num_cores=2, num_subcores=16)

    @functools.partial(
        pl.kernel,
        mesh=mesh,
        out_type=jax.ShapeDtypeStruct((_B, _D), jnp.float32),
        scratch_types=[
            pltpu.VMEM((_ROWS_PER_W * 3,), jnp.int32),
            pltpu.VMEM((2, 3 * _G, _WPR), jnp.int32),
            pltpu.VMEM((2, _HALF, _D), jnp.float32),
            pltpu.SemaphoreType.DMA,
            pltpu.SemaphoreType.DMA,
            pltpu.SemaphoreType.DMA,
        ],
    )
    def k(pk_hbm, idx_hbm, out_hbm, idx_v, gbuf, obuf, gsem, osem0, osem1):
        wid = lax.axis_index("s") * 2 + lax.axis_index("c")
        base = wid * _ROWS_PER_W
        pltpu.sync_copy(idx_hbm.at[pl.ds(base * 3, _ROWS_PER_W * 3)], idx_v)

        def start_gather(g, par):
            pltpu.async_copy(
                pk_hbm.at[idx_v.at[pl.ds(g * (3 * _G), 3 * _G)]],
                gbuf.at[par], gsem)

        start_gather(0, 0)

        sign_mask = jnp.full((16,), np.int32(-2147483648), jnp.int32)
        one_bits = jnp.full((16,), np.int32(0x3F800000), jnp.int32)

        def group(g, carry):
            par = lax.rem(g, 2)
            pltpu.make_async_copy(
                pk_hbm.at[idx_v.at[pl.ds(0, 3 * _G)]],
                gbuf.at[par], gsem).wait()

            @pl.when(g + 1 < _NGROUPS)
            def _():
                start_gather(g + 1, 1 - par)

            for h, osem in ((0, osem0), (1, osem1)):
                @pl.when(g > 0)
                def _(h=h, osem=osem):
                    pltpu.make_async_copy(
                        obuf.at[h],
                        out_hbm.at[pl.ds(base, _HALF)], osem).wait()

                def row(pp, c2, h=h):
                    p = h * _HALF + pp

                    def wordblk(wi, c3, p=p):
                        a = gbuf[par, 3 * p, pl.ds(wi * 16, 16)]
                        b = gbuf[par, 3 * p + 1, pl.ds(wi * 16, 16)]
                        d = gbuf[par, 3 * p + 2, pl.ds(wi * 16, 16)]
                        wv = lax.bitwise_xor(lax.bitwise_xor(a, b), d)
                        for kk in range(32):
                            s = lax.bitwise_and(
                                lax.shift_left(wv, jnp.full((16,), 31 - kk,
                                                            jnp.int32)),
                                sign_mask)
                            val = lax.bitcast_convert_type(
                                lax.bitwise_or(s, one_bits), jnp.float32)
                            obuf[h, pp, pl.ds(wi * 512 + kk * 16, 16)] = val
                        return c3

                    lax.fori_loop(0, _WPR // 16, wordblk, 0)
                    return c2

                lax.fori_loop(0, _HALF, row, 0)
                pltpu.async_copy(
                    obuf.at[h],
                    out_hbm.at[pl.ds(base + g * _G + h * _HALF, _HALF)], osem)
            return carry

        lax.fori_loop(0, _NGROUPS, group, 0)
        for h, osem in ((0, osem0), (1, osem1)):
            pltpu.make_async_copy(
                obuf.at[h], out_hbm.at[pl.ds(base, _HALF)], osem).wait()

    return k(pk_tab, idx_flat)


def kernel(x, level_weight):
    pk_tab, idx_flat = _prep(x, level_weight)
    return _sc_encode(pk_tab, idx_flat)


# speedup vs baseline: 8.6958x; 1.0709x over previous
import functools

import numpy as np
import jax
import jax.numpy as jnp
from jax import lax
from jax.experimental import pallas as pl
from jax.experimental.pallas import tpu as pltpu
from jax.experimental.pallas import tpu_sc as plsc

_LEVELS = 256
_D = 8192
_B = 4096
_NW = 32
_ROWS_PER_W = _B // _NW
_WPR = _D // 32
_G = 8
_NGROUPS = _ROWS_PER_W // _G
_HALF = _G // 2


def _prep(x, w):
    def body(x_ref, w_ref, pk_ref, idx_ref):
        wv = w_ref[...]
        for c, sh in enumerate((2, 1, 0)):
            bc = wv if sh == 0 else jnp.concatenate(
                [wv[:, -sh:], wv[:, :-sh]], axis=1)
            bits = lax.shift_right_logical(
                lax.bitcast_convert_type(bc, jnp.int32), 31)
            word = bits[:, 0:_WPR]
            for k in range(1, 32):
                word = word | (bits[:, k * _WPR:(k + 1) * _WPR] << k)
            pk_ref[pl.ds(c * _LEVELS, _LEVELS), :] = word

        xv = x_ref[...]
        q = jnp.clip(jnp.round(xv * (_LEVELS - 1)).astype(jnp.int32),
                     0, _LEVELS - 1)
        ch = lax.broadcasted_iota(jnp.int32, (_B, 3), 1)
        idx_ref[...] = q + ch * _LEVELS

    pk, idx = pl.pallas_call(
        body,
        out_shape=[
            jax.ShapeDtypeStruct((3 * _LEVELS, _WPR), jnp.int32),
            jax.ShapeDtypeStruct((_B, 3), jnp.int32),
        ],
    )(x, w)
    return pk, idx.reshape(-1)


def _sc_encode(pk_tab, idx_flat):
    mesh = plsc.VectorSubcoreMesh(
        core_axis_name="c", subcore_axis_name="s", num_cores=2, num_subcores=16)

    @functools.partial(
        pl.kernel,
        mesh=mesh,
        out_type=jax.ShapeDtypeStruct((_B, _D), jnp.float32),
        scratch_types=[
            pltpu.VMEM((_ROWS_PER_W * 3,), jnp.int32),
            pltpu.VMEM((2, 3 * _G, _WPR), jnp.int32),
            pltpu.VMEM((2, _HALF, _D), jnp.float32),
            pltpu.SemaphoreType.DMA,
            pltpu.SemaphoreType.DMA,
            pltpu.SemaphoreType.DMA,
        ],
    )
    def k(pk_hbm, idx_hbm, out_hbm, idx_v, gbuf, obuf, gsem, osem0, osem1):
        wid = lax.axis_index("s") * 2 + lax.axis_index("c")
        base = wid * _ROWS_PER_W
        pltpu.sync_copy(idx_hbm.at[pl.ds(base * 3, _ROWS_PER_W * 3)], idx_v)

        def start_gather(g, par):
            pltpu.async_copy(
                pk_hbm.at[idx_v.at[pl.ds(g * (3 * _G), 3 * _G)]],
                gbuf.at[par], gsem)

        start_gather(0, 0)

        sign_mask = jnp.full((16,), np.int32(-2147483648), jnp.int32)
        one_bits = jnp.full((16,), np.int32(0x3F800000), jnp.int32)

        def group(g, carry):
            par = lax.rem(g, 2)
            pltpu.make_async_copy(
                pk_hbm.at[idx_v.at[pl.ds(0, 3 * _G)]],
                gbuf.at[par], gsem).wait()

            @pl.when(g + 1 < _NGROUPS)
            def _():
                start_gather(g + 1, 1 - par)

            for h, osem in ((0, osem0), (1, osem1)):
                @pl.when(g > 0)
                def _(h=h, osem=osem):
                    pltpu.make_async_copy(
                        obuf.at[h],
                        out_hbm.at[pl.ds(base, _HALF)], osem).wait()

                def row(pp, c2, h=h):
                    p = h * _HALF + pp

                    def wordblk(wi, c3, p=p):
                        a = gbuf[par, 3 * p, pl.ds(wi * 16, 16)]
                        b = gbuf[par, 3 * p + 1, pl.ds(wi * 16, 16)]
                        d = gbuf[par, 3 * p + 2, pl.ds(wi * 16, 16)]
                        wv = lax.bitwise_xor(lax.bitwise_xor(a, b), d)
                        for kk in range(32):
                            s = lax.bitwise_and(
                                lax.shift_left(wv, jnp.full((16,), 31 - kk,
                                                            jnp.int32)),
                                sign_mask)
                            val = lax.bitcast_convert_type(
                                lax.bitwise_or(s, one_bits), jnp.float32)
                            obuf[h, pp, pl.ds(kk * _WPR + wi * 16, 16)] = val
                        return c3

                    lax.fori_loop(0, _WPR // 16, wordblk, 0)
                    return c2

                lax.fori_loop(0, _HALF, row, 0)
                pltpu.async_copy(
                    obuf.at[h],
                    out_hbm.at[pl.ds(base + g * _G + h * _HALF, _HALF)], osem)
            return carry

        lax.fori_loop(0, _NGROUPS, group, 0)
        for h, osem in ((0, osem0), (1, osem1)):
            pltpu.make_async_copy(
                obuf.at[h], out_hbm.at[pl.ds(base, _HALF)], osem).wait()

    return k(pk_tab, idx_flat)


def kernel(x, level_weight):
    pk_tab, idx_flat = _prep(x, level_weight)
    return _sc_encode(pk_tab, idx_flat)
